# baseline (speedup 1.0000x reference)
import functools

import jax
import jax.numpy as jnp
from jax import lax
from jax.experimental import pallas as pl
from jax.experimental.pallas import tpu as pltpu
from jax.experimental.pallas import tpu_sc as plsc

EB = 512
NC, NS = 2, 16
NW = NC * NS


def _agg_body(idx_b, idx_f, t_f, raw_f, wmsg, bmsg, mean_o, tmax_o):
    nblk = idx_f.shape[0]
    idx_i = idx_b[0, 0, :]
    acc = jnp.zeros((EB, 17), jnp.float32)
    tmax = jnp.full((EB,), -jnp.inf, jnp.float32)
    ones = jnp.ones((EB, 1), jnp.bfloat16)
    for j in range(nblk):
        idx_j = idx_f[j, :]
        t_j = t_f[j, :]
        raw_j = raw_f[j * EB:(j + 1) * EB, :]
        msg_j = jax.nn.relu(
            jax.lax.dot(raw_j, wmsg[...], preferred_element_type=jnp.float32)
            + bmsg[0, :][None, :])
        rhs_j = jnp.concatenate([msg_j.astype(jnp.bfloat16), ones], axis=1)
        eq = idx_i[:, None] == idx_j[None, :]
        match_b = jnp.where(eq, 1.0, 0.0).astype(jnp.bfloat16)
        acc += jax.lax.dot(match_b, rhs_j, preferred_element_type=jnp.float32)
        tm = jnp.where(eq, t_j[None, :], -jnp.inf)
        tmax = jnp.maximum(tmax, jnp.max(tm, axis=1))
    counts = jnp.maximum(acc[:, 16:17], 1.0)
    mean_o[...] = acc[:, :16] / counts
    tmax_o[0, 0, :] = tmax


def _aggregate(idx, t, raw_msg, W_msg, b_msg):
    E = idx.shape[0]
    nblk = E // EB
    idx3 = idx.reshape(nblk, 1, EB)
    idx2 = idx.reshape(nblk, EB)
    t2 = t.reshape(nblk, EB)
    full = lambda shp: pl.BlockSpec(shp, lambda i: (0,) * len(shp))
    mean, tmax3 = pl.pallas_call(
        _agg_body,
        grid=(nblk,),
        in_specs=[
            pl.BlockSpec((1, 1, EB), lambda i: (i, 0, 0)),
            full((nblk, EB)),
            full((nblk, EB)),
            full((E, raw_msg.shape[1])),
            full(W_msg.shape),
            full((1, 16)),
        ],
        out_specs=[
            pl.BlockSpec((EB, 16), lambda i: (i, 0)),
            pl.BlockSpec((1, 1, EB), lambda i: (i, 0, 0)),
        ],
        out_shape=[
            jax.ShapeDtypeStruct((E, 16), jnp.float32),
            jax.ShapeDtypeStruct((nblk, 1, EB), jnp.float32),
        ],
        compiler_params=pltpu.CompilerParams(
            dimension_semantics=("parallel",)),
    )(idx3, idx2, t2, raw_msg, W_msg, b_msg.reshape(1, 16))
    return mean, tmax3.reshape(E)


def _gru_body(m, old, wir, wiz, win, whr, whz, whn, bi, bh, out):
    dot = functools.partial(jax.lax.dot, preferred_element_type=jnp.float32)
    mv, ov = m[...], old[...]
    r = jax.nn.sigmoid(dot(mv, wir[...]) + bi[0, 0:16][None, :]
                       + dot(ov, whr[...]) + bh[0, 0:16][None, :])
    z = jax.nn.sigmoid(dot(mv, wiz[...]) + bi[0, 16:32][None, :]
                       + dot(ov, whz[...]) + bh[0, 16:32][None, :])
    n = jnp.tanh(dot(mv, win[...]) + bi[0, 32:48][None, :]
                 + r * (dot(ov, whn[...]) + bh[0, 32:48][None, :]))
    out[...] = (1.0 - z) * n + z * ov


def _gru(mean, old, W_i, W_h, b_i, b_h):
    E = mean.shape[0]
    nblk = E // EB
    full = lambda shp: pl.BlockSpec(shp, lambda i: (0,) * len(shp))
    row = pl.BlockSpec((EB, 16), lambda i: (i, 0))
    return pl.pallas_call(
        _gru_body,
        grid=(nblk,),
        in_specs=[row, row] + [full((16, 16))] * 6 + [full((1, 48))] * 2,
        out_specs=row,
        out_shape=jax.ShapeDtypeStruct((E, 16), jnp.float32),
        compiler_params=pltpu.CompilerParams(
            dimension_semantics=("parallel",)),
    )(mean, old,
      W_i[:, 0:16], W_i[:, 16:32], W_i[:, 32:48],
      W_h[:, 0:16], W_h[:, 16:32], W_h[:, 32:48],
      b_i.reshape(1, 48), b_h.reshape(1, 48))


def _copy_body(mem_i, lu_i, mem_o, lu_o):
    mem_o[...] = mem_i[...]
    lu_o[...] = lu_i[...]


def _copy_tables(memory, last_update):
    N, D = memory.shape
    rows = N * D // 128
    mem2 = memory.reshape(rows, 128)
    lu2 = last_update.reshape(N // 125, 125)
    g = 25
    mem_c, lu_c = pl.pallas_call(
        _copy_body,
        grid=(g,),
        in_specs=[pl.BlockSpec((rows // g, 128), lambda i: (i, 0)),
                  pl.BlockSpec((N // 125 // g, 125), lambda i: (i, 0))],
        out_specs=[pl.BlockSpec((rows // g, 128), lambda i: (i, 0)),
                   pl.BlockSpec((N // 125 // g, 125), lambda i: (i, 0))],
        out_shape=[jax.ShapeDtypeStruct((rows, 128), jnp.float32),
                   jax.ShapeDtypeStruct((N // 125, 125), jnp.float32)],
        compiler_params=pltpu.CompilerParams(
            dimension_semantics=("parallel",)),
    )(mem2, lu2)
    return mem_c.reshape(N, D), lu_c.reshape(N)


def _sc_mesh():
    return plsc.VectorSubcoreMesh(
        core_axis_name="c", subcore_axis_name="s",
        num_cores=NC, num_subcores=NS)


_SC_PARAMS = pltpu.CompilerParams(use_tc_tiling_on_sc=False)


def _gather_rows(memory, idx):
    E = idx.shape[0]
    D = memory.shape[1]

    @functools.partial(
        pl.kernel, mesh=_sc_mesh(), compiler_params=_SC_PARAMS,
        out_type=jax.ShapeDtypeStruct((E, D), jnp.float32),
        scratch_types=[pltpu.VMEM((EB,), jnp.int32),
                       pltpu.VMEM((EB, D), jnp.float32),
                       pltpu.SemaphoreType.DMA])
    def gat(mem_hbm, idx_hbm, out_hbm, idx_v, rows_v, sem):
        wid = lax.axis_index("s") * NC + lax.axis_index("c")
        base = wid * EB
        pltpu.sync_copy(idx_hbm.at[pl.ds(base, EB)], idx_v)
        pltpu.async_copy(mem_hbm.at[idx_v], rows_v, sem).wait()
        pltpu.sync_copy(rows_v, out_hbm.at[pl.ds(base, EB)])

    return gat(memory, idx)


def _copy_scatter(memory, last_update, new_mem, tmax, idx):
    E = idx.shape[0]
    N, D = memory.shape
    SB = E // NS
    RB = (N // NS) // 8 * 8
    TAIL = N - NS * RB

    @functools.partial(
        pl.kernel,
        mesh=plsc.VectorSubcoreMesh(core_axis_name="c", subcore_axis_name="s",
                                    num_cores=1, num_subcores=NS),
        out_type=(jax.ShapeDtypeStruct((N, D), jnp.float32),
                  jax.ShapeDtypeStruct((N,), jnp.float32)),
        compiler_params=_SC_PARAMS,
        scratch_types=[pltpu.VMEM((SB,), jnp.int32),
                       pltpu.VMEM((SB, D), jnp.float32),
                       pltpu.VMEM((SB,), jnp.float32),
                       pltpu.SemaphoreType.DMA])
    def cs(mem_hbm, lu_hbm, new_hbm, tmax_hbm, idx_hbm, mem_o, lu_o,
           idx_v, rows_v, t_v, sem):
        wid = lax.axis_index("s")
        rbase = wid * RB
        pltpu.sync_copy(mem_hbm.at[pl.ds(rbase, RB)], mem_o.at[pl.ds(rbase, RB)])
        pltpu.sync_copy(lu_hbm.at[pl.ds(rbase, RB)], lu_o.at[pl.ds(rbase, RB)])

        @pl.when(wid == NS - 1)
        def _():
            tb = NS * RB
            pltpu.sync_copy(mem_hbm.at[pl.ds(tb, TAIL)],
                            mem_o.at[pl.ds(tb, TAIL)])
            pltpu.sync_copy(lu_hbm.at[pl.ds(tb, TAIL)],
                            lu_o.at[pl.ds(tb, TAIL)])

        plsc.subcore_barrier()
        base = wid * SB
        pltpu.sync_copy(idx_hbm.at[pl.ds(base, SB)], idx_v)
        pltpu.async_copy(new_hbm.at[pl.ds(base, SB)], rows_v, sem).wait()
        pltpu.sync_copy(rows_v, mem_o.at[idx_v])
        pltpu.async_copy(tmax_hbm.at[pl.ds(base, SB)], t_v, sem).wait()
        pltpu.sync_copy(t_v, lu_o.at[idx_v])

    return cs(memory, last_update, new_mem, tmax, idx)


def kernel(memory, last_update, raw_msg, t, idx, W_msg, b_msg, W_i, W_h,
           b_i, b_h):
    mean, tmax = _aggregate(idx, t, raw_msg, W_msg, b_msg)
    old = _gather_rows(memory, idx)
    new_mem = _gru(mean, old, W_i, W_h, b_i, b_h)
    return _copy_scatter(memory, last_update, new_mem, tmax, idx)


# speedup vs baseline: 2.8825x; 2.8825x over previous
import functools

import jax
import jax.numpy as jnp
from jax import lax
from jax.experimental import pallas as pl
from jax.experimental.pallas import tpu as pltpu
from jax.experimental.pallas import tpu_sc as plsc

EB = 512
NC, NS = 2, 16
NW = NC * NS


def _msg_body(raw, wmsg, bmsg, rhs_o):
    m = jax.nn.relu(
        jax.lax.dot(raw[...], wmsg[...], preferred_element_type=jnp.float32)
        + bmsg[0, :][None, :])
    E = m.shape[0]
    rhs_o[...] = jnp.concatenate(
        [m.astype(jnp.bfloat16), jnp.ones((E, 1), jnp.bfloat16),
         jnp.zeros((E, 15), jnp.bfloat16)], axis=1)


def _messages(raw_msg, W_msg, b_msg):
    E = raw_msg.shape[0]
    full = lambda shp: pl.BlockSpec(shp, lambda: (0,) * len(shp))
    return pl.pallas_call(
        _msg_body,
        in_specs=[full((E, raw_msg.shape[1])), full(W_msg.shape),
                  full((1, 16))],
        out_specs=full((E, 32)),
        out_shape=jax.ShapeDtypeStruct((E, 32), jnp.bfloat16),
    )(raw_msg, W_msg, b_msg.reshape(1, 16))


def _agg_body(idx_b, idx_f, t_f, rhs_f, mean_o, tmax_o):
    nblk = idx_f.shape[0]
    idx_i = idx_b[0, 0, :]
    acc = jnp.zeros((EB, 32), jnp.float32)
    tmax = jnp.full((EB,), -jnp.inf, jnp.float32)
    for j in range(nblk):
        idx_j = idx_f[j, :]
        t_j = t_f[j, :]
        rhs_j = rhs_f[j * EB:(j + 1) * EB, :]
        eq = idx_i[:, None] == idx_j[None, :]
        match_b = jnp.where(eq, 1.0, 0.0).astype(jnp.bfloat16)
        acc += jax.lax.dot(match_b, rhs_j, preferred_element_type=jnp.float32)
        tm = jnp.where(eq, t_j[None, :], -jnp.inf)
        tmax = jnp.maximum(tmax, jnp.max(tm, axis=1))
    counts = jnp.maximum(acc[:, 16:17], 1.0)
    mean_o[...] = acc[:, :16] / counts
    tmax_o[0, 0, :] = tmax


def _aggregate(idx, t, raw_msg, W_msg, b_msg):
    E = idx.shape[0]
    rhs = _messages(raw_msg, W_msg, b_msg)
    nblk = E // EB
    idx3 = idx.reshape(nblk, 1, EB)
    idx2 = idx.reshape(nblk, EB)
    t2 = t.reshape(nblk, EB)
    full = lambda shp: pl.BlockSpec(shp, lambda i: (0,) * len(shp))
    mean, tmax3 = pl.pallas_call(
        _agg_body,
        grid=(nblk,),
        in_specs=[
            pl.BlockSpec((1, 1, EB), lambda i: (i, 0, 0)),
            full((nblk, EB)),
            full((nblk, EB)),
            full((E, 32)),
        ],
        out_specs=[
            pl.BlockSpec((EB, 16), lambda i: (i, 0)),
            pl.BlockSpec((1, 1, EB), lambda i: (i, 0, 0)),
        ],
        out_shape=[
            jax.ShapeDtypeStruct((E, 16), jnp.float32),
            jax.ShapeDtypeStruct((nblk, 1, EB), jnp.float32),
        ],
        compiler_params=pltpu.CompilerParams(
            dimension_semantics=("parallel",)),
    )(idx3, idx2, t2, rhs)
    return mean, tmax3.reshape(E)


def _gru_body(m, old, wir, wiz, win, whr, whz, whn, bi, bh, out):
    dot = functools.partial(jax.lax.dot, preferred_element_type=jnp.float32)
    mv, ov = m[...], old[...]
    r = jax.nn.sigmoid(dot(mv, wir[...]) + bi[0, 0:16][None, :]
                       + dot(ov, whr[...]) + bh[0, 0:16][None, :])
    z = jax.nn.sigmoid(dot(mv, wiz[...]) + bi[0, 16:32][None, :]
                       + dot(ov, whz[...]) + bh[0, 16:32][None, :])
    n = jnp.tanh(dot(mv, win[...]) + bi[0, 32:48][None, :]
                 + r * (dot(ov, whn[...]) + bh[0, 32:48][None, :]))
    out[...] = (1.0 - z) * n + z * ov


def _gru(mean, old, W_i, W_h, b_i, b_h):
    E = mean.shape[0]
    nblk = E // EB
    full = lambda shp: pl.BlockSpec(shp, lambda i: (0,) * len(shp))
    row = pl.BlockSpec((EB, 16), lambda i: (i, 0))
    return pl.pallas_call(
        _gru_body,
        grid=(nblk,),
        in_specs=[row, row] + [full((16, 16))] * 6 + [full((1, 48))] * 2,
        out_specs=row,
        out_shape=jax.ShapeDtypeStruct((E, 16), jnp.float32),
        compiler_params=pltpu.CompilerParams(
            dimension_semantics=("parallel",)),
    )(mean, old,
      W_i[:, 0:16], W_i[:, 16:32], W_i[:, 32:48],
      W_h[:, 0:16], W_h[:, 16:32], W_h[:, 32:48],
      b_i.reshape(1, 48), b_h.reshape(1, 48))


def _copy_body(mem_i, lu_i, mem_o, lu_o):
    mem_o[...] = mem_i[...]
    lu_o[...] = lu_i[...]


def _copy_tables(memory, last_update):
    N, D = memory.shape
    rows = N * D // 128
    mem2 = memory.reshape(rows, 128)
    lu2 = last_update.reshape(N // 125, 125)
    g = 25
    mem_c, lu_c = pl.pallas_call(
        _copy_body,
        grid=(g,),
        in_specs=[pl.BlockSpec((rows // g, 128), lambda i: (i, 0)),
                  pl.BlockSpec((N // 125 // g, 125), lambda i: (i, 0))],
        out_specs=[pl.BlockSpec((rows // g, 128), lambda i: (i, 0)),
                   pl.BlockSpec((N // 125 // g, 125), lambda i: (i, 0))],
        out_shape=[jax.ShapeDtypeStruct((rows, 128), jnp.float32),
                   jax.ShapeDtypeStruct((N // 125, 125), jnp.float32)],
        compiler_params=pltpu.CompilerParams(
            dimension_semantics=("parallel",)),
    )(mem2, lu2)
    return mem_c.reshape(N, D), lu_c.reshape(N)


def _sc_mesh():
    return plsc.VectorSubcoreMesh(
        core_axis_name="c", subcore_axis_name="s",
        num_cores=NC, num_subcores=NS)


_SC_PARAMS = pltpu.CompilerParams(use_tc_tiling_on_sc=False)


def _gather_rows(memory, idx):
    E = idx.shape[0]
    D = memory.shape[1]

    @functools.partial(
        pl.kernel, mesh=_sc_mesh(), compiler_params=_SC_PARAMS,
        out_type=jax.ShapeDtypeStruct((E, D), jnp.float32),
        scratch_types=[pltpu.VMEM((EB,), jnp.int32),
                       pltpu.VMEM((EB, D), jnp.float32),
                       pltpu.SemaphoreType.DMA])
    def gat(mem_hbm, idx_hbm, out_hbm, idx_v, rows_v, sem):
        wid = lax.axis_index("s") * NC + lax.axis_index("c")
        base = wid * EB
        pltpu.sync_copy(idx_hbm.at[pl.ds(base, EB)], idx_v)
        pltpu.async_copy(mem_hbm.at[idx_v], rows_v, sem).wait()
        pltpu.sync_copy(rows_v, out_hbm.at[pl.ds(base, EB)])

    return gat(memory, idx)


def _scatter_rows(mem_ref, lu_ref, new_mem, tmax, idx):
    E = idx.shape[0]
    D = new_mem.shape[1]

    @functools.partial(
        pl.kernel, mesh=_sc_mesh(), out_type=(), compiler_params=_SC_PARAMS,
        scratch_types=[pltpu.VMEM((EB,), jnp.int32),
                       pltpu.VMEM((EB, D), jnp.float32),
                       pltpu.VMEM((EB,), jnp.float32),
                       pltpu.SemaphoreType.DMA])
    def scat(mem_hbm, lu_hbm, new_hbm, tmax_hbm, idx_hbm,
             idx_v, rows_v, t_v, sem):
        wid = lax.axis_index("s") * NC + lax.axis_index("c")
        base = wid * EB
        pltpu.sync_copy(idx_hbm.at[pl.ds(base, EB)], idx_v)
        pltpu.async_copy(new_hbm.at[pl.ds(base, EB)], rows_v, sem).wait()
        pltpu.sync_copy(rows_v, mem_hbm.at[idx_v])
        pltpu.async_copy(tmax_hbm.at[pl.ds(base, EB)], t_v, sem).wait()
        pltpu.sync_copy(t_v, lu_hbm.at[idx_v])

    scat(mem_ref, lu_ref, new_mem, tmax, idx)


def kernel(memory, last_update, raw_msg, t, idx, W_msg, b_msg, W_i, W_h,
           b_i, b_h):
    mem_ref = jax.new_ref(memory)
    lu_ref = jax.new_ref(last_update)
    mean, tmax = _aggregate(idx, t, raw_msg, W_msg, b_msg)
    old = _gather_rows(memory, idx)
    new_mem = _gru(mean, old, W_i, W_h, b_i, b_h)
    _scatter_rows(mem_ref, lu_ref, new_mem, tmax, idx)
    return mem_ref[...], lu_ref[...]
